# Initial kernel scaffold; baseline (speedup 1.0000x reference)
import jax
import jax.numpy as jnp
from jax.experimental import pallas as pl


def kernel(seg_map, feature_map):
    raise NotImplementedError("write your pallas kernel here")


# speedup vs baseline: 1.4054x; 1.4054x over previous
import functools

import jax
import jax.numpy as jnp
from jax import lax
from jax.experimental import pallas as pl
from jax.experimental.pallas import tpu as pltpu
from jax.experimental.pallas import tpu_sc as plsc

H = 512
W = 512
P = H * W
R = 2048
D = 256
TPAD = 2056
CB = 16
NCB = D // CB
G = 1024
L = 16


def _transpose_pad(fm):
    def body(f_ref, o_ref):
        t = f_ref[...].T
        o_ref[...] = jnp.concatenate(
            [t, jnp.zeros((D, TPAD - R), jnp.float32)], axis=1
        )

    return pl.pallas_call(
        body,
        out_shape=jax.ShapeDtypeStruct((D, TPAD), jnp.float32),
    )(fm)


def _sc_gather(ftT, seg):
    info = plsc.get_sparse_core_info()
    NC, NS = info.num_cores, info.num_subcores
    NW = NC * NS
    PW = P // NW
    NG = PW // G

    mesh = plsc.VectorSubcoreMesh(core_axis_name="c", subcore_axis_name="s")

    @functools.partial(
        pl.kernel,
        mesh=mesh,
        compiler_params=pltpu.CompilerParams(needs_layout_passes=False),
        out_type=[
            jax.ShapeDtypeStruct((D, P), jnp.float32),
            jax.ShapeDtypeStruct((P,), jnp.float32),
        ],
        scratch_types=[
            pltpu.VMEM((PW,), jnp.int32),
            pltpu.VMEM((PW,), jnp.float32),
            pltpu.VMEM((CB * TPAD,), jnp.float32),
            pltpu.VMEM((CB, G), jnp.float32),
        ],
    )
    def k(ftT_hbm, seg_hbm, out_hbm, mask_hbm, idx_v, val_v, tab_v, obuf_v):
        wid = lax.axis_index("s") * NC + lax.axis_index("c")
        base = wid * PW

        pltpu.sync_copy(seg_hbm.at[pl.ds(base, PW)], idx_v)

        def mk(i, carry):
            s = idx_v[pl.ds(i * L, L)]
            valid = (s >= 0) & (s < R)
            idx_v[pl.ds(i * L, L)] = jnp.where(valid, s, R)
            val_v[pl.ds(i * L, L)] = jnp.where(valid, 1.0, 0.0).astype(jnp.float32)
            return carry

        lax.fori_loop(0, PW // L, mk, 0)
        pltpu.sync_copy(val_v, mask_hbm.at[pl.ds(base, PW)])

        def cb_body(cb, carry):
            pltpu.sync_copy(ftT_hbm.at[pl.ds(cb * (CB * TPAD), CB * TPAD)], tab_v)

            def g_body(g, carry2):
                def step(i, carry3):
                    iv = idx_v[pl.ds(g * G + i * L, L)]
                    for c in range(CB):
                        vals = plsc.load_gather(tab_v, [iv + (c * TPAD)])
                        obuf_v[c, pl.ds(i * L, L)] = vals
                    return carry3

                lax.fori_loop(0, G // L, step, 0)
                pltpu.sync_copy(
                    obuf_v,
                    out_hbm.at[pl.ds(cb * CB, CB), pl.ds(base + g * G, G)],
                )
                return carry2

            lax.fori_loop(0, NG, g_body, 0)
            return carry

        lax.fori_loop(0, NCB, cb_body, 0)

    return k(ftT, seg)


def kernel(seg_map, feature_map):
    ftT = _transpose_pad(feature_map).reshape(-1)
    out, mask = _sc_gather(ftT, seg_map.reshape(-1))
    return out.reshape(D, H, W), mask.reshape(1, H, W)


# speedup vs baseline: 1.5368x; 1.0934x over previous
import functools

import jax
import jax.numpy as jnp
from jax import lax
from jax.experimental import pallas as pl
from jax.experimental.pallas import tpu as pltpu
from jax.experimental.pallas import tpu_sc as plsc

H = 512
W = 512
P = H * W
R = 2048
D = 256
TPAD = 2056
CB = 16
NCB = D // CB
G = 1024
L = 16


def _transpose_pad(fm):
    def body(f_ref, o_ref):
        t = f_ref[...].T
        o_ref[...] = jnp.concatenate(
            [t, jnp.zeros((D, TPAD - R), jnp.float32)], axis=1
        )

    return pl.pallas_call(
        body,
        out_shape=jax.ShapeDtypeStruct((D, TPAD), jnp.float32),
    )(fm)


def _sc_gather(ftT, seg):
    info = plsc.get_sparse_core_info()
    NC, NS = info.num_cores, info.num_subcores
    NW = NC * NS
    PW = P // NW
    NG = PW // G

    mesh = plsc.VectorSubcoreMesh(core_axis_name="c", subcore_axis_name="s")

    @functools.partial(
        pl.kernel,
        mesh=mesh,
        compiler_params=pltpu.CompilerParams(needs_layout_passes=False),
        out_type=[
            jax.ShapeDtypeStruct((D, P), jnp.float32),
            jax.ShapeDtypeStruct((P,), jnp.float32),
        ],
        scratch_types=[
            pltpu.VMEM((PW,), jnp.int32),
            pltpu.VMEM((PW,), jnp.float32),
            pltpu.VMEM((CB * TPAD,), jnp.float32),
            pltpu.VMEM((CB, G), jnp.float32),
            pltpu.VMEM((CB, G), jnp.float32),
            pltpu.SemaphoreType.DMA,
            pltpu.SemaphoreType.DMA,
        ],
    )
    def k(ftT_hbm, seg_hbm, out_hbm, mask_hbm, idx_v, val_v, tab_v,
          obA, obB, semA, semB):
        wid = lax.axis_index("s") * NC + lax.axis_index("c")
        base = wid * PW

        pltpu.sync_copy(seg_hbm.at[pl.ds(base, PW)], idx_v)

        def mk(i, carry):
            s = idx_v[pl.ds(i * L, L)]
            valid = (s >= 0) & (s < R)
            idx_v[pl.ds(i * L, L)] = jnp.where(valid, s, R)
            val_v[pl.ds(i * L, L)] = jnp.where(valid, 1.0, 0.0).astype(jnp.float32)
            return carry

        lax.fori_loop(0, PW // L, mk, 0)
        pltpu.sync_copy(val_v, mask_hbm.at[pl.ds(base, PW)])

        def fill(obuf, g):
            def step(i, carry3):
                iv = idx_v[pl.ds(g * G + i * L, L)]
                for c in range(CB):
                    vals = plsc.load_gather(tab_v, [iv + (c * TPAD)])
                    obuf[c, pl.ds(i * L, L)] = vals
                return carry3

            lax.fori_loop(0, G // L, step, 0)

        def out_slab(cb, g):
            return out_hbm.at[pl.ds(cb * CB, CB), pl.ds(base + g * G, G)]

        def cb_body(cb, carry):
            pltpu.sync_copy(ftT_hbm.at[pl.ds(cb * (CB * TPAD), CB * TPAD)], tab_v)

            def gp_body(gp, carry2):
                g0 = 2 * gp
                g1 = 2 * gp + 1

                @pl.when(gp > 0)
                def _():
                    pltpu.make_async_copy(obA, out_slab(cb, g0), semA).wait()

                fill(obA, g0)
                pltpu.async_copy(obA, out_slab(cb, g0), semA)

                @pl.when(gp > 0)
                def _():
                    pltpu.make_async_copy(obB, out_slab(cb, g1), semB).wait()

                fill(obB, g1)
                pltpu.async_copy(obB, out_slab(cb, g1), semB)
                return carry2

            lax.fori_loop(0, NG // 2, gp_body, 0)
            pltpu.make_async_copy(obA, out_slab(cb, 0), semA).wait()
            pltpu.make_async_copy(obB, out_slab(cb, 0), semB).wait()
            return carry

        lax.fori_loop(0, NCB, cb_body, 0)

    return k(ftT, seg)


def kernel(seg_map, feature_map):
    ftT = _transpose_pad(feature_map).reshape(-1)
    out, mask = _sc_gather(ftT, seg_map.reshape(-1))
    return out.reshape(D, H, W), mask.reshape(1, H, W)


# speedup vs baseline: 3.1921x; 2.0772x over previous
import functools

import jax
import jax.numpy as jnp
from jax import lax
from jax.experimental import pallas as pl
from jax.experimental.pallas import tpu as pltpu
from jax.experimental.pallas import tpu_sc as plsc

H = 512
W = 512
P = H * W
R = 2048
D = 256
TPAD = 2056
CB = 16
NCB = D // CB
G = 1024
L = 16


def _transpose_pad(fm):
    def body(f_ref, o_ref):
        t = f_ref[...].T
        o_ref[...] = jnp.concatenate(
            [t, jnp.zeros((D, TPAD - R), jnp.float32)], axis=1
        )

    return pl.pallas_call(
        body,
        out_shape=jax.ShapeDtypeStruct((D, TPAD), jnp.float32),
    )(fm)


def _sc_gather(ftT, seg):
    info = plsc.get_sparse_core_info()
    NC, NS = info.num_cores, info.num_subcores
    NW = NC * NS
    PW = P // NW
    NG = PW // G

    mesh = plsc.VectorSubcoreMesh(core_axis_name="c", subcore_axis_name="s")

    @functools.partial(
        pl.kernel,
        mesh=mesh,
        compiler_params=pltpu.CompilerParams(needs_layout_passes=False),
        out_type=[
            jax.ShapeDtypeStruct((D, P), jnp.float32),
            jax.ShapeDtypeStruct((P,), jnp.float32),
        ],
        scratch_types=[
            pltpu.VMEM((PW,), jnp.int32),
            pltpu.VMEM((PW,), jnp.float32),
            pltpu.VMEM((CB * TPAD,), jnp.float32),
            pltpu.VMEM((CB, G), jnp.float32),
            pltpu.VMEM((CB, G), jnp.float32),
            pltpu.SemaphoreType.DMA,
            pltpu.SemaphoreType.DMA,
        ],
    )
    def k(ftT_hbm, seg_hbm, out_hbm, mask_hbm, idx_v, val_v, tab_v,
          obA, obB, semA, semB):
        wid = lax.axis_index("s") * NC + lax.axis_index("c")
        base = wid * PW

        pltpu.sync_copy(seg_hbm.at[pl.ds(base, PW)], idx_v)

        @plsc.parallel_loop(0, PW, step=L, unroll=4)
        def _(p):
            s = idx_v[pl.ds(p, L)]
            valid = (s >= 0) & (s < R)
            idx_v[pl.ds(p, L)] = jnp.where(valid, s, R)
            val_v[pl.ds(p, L)] = jnp.where(valid, 1.0, 0.0).astype(jnp.float32)
        pltpu.sync_copy(val_v, mask_hbm.at[pl.ds(base, PW)])

        def fill(obuf, g):
            @plsc.parallel_loop(0, G, step=L, unroll=2)
            def _(p):
                iv = idx_v[pl.ds(g * G + p, L)]
                for c in range(CB):
                    vals = plsc.load_gather(tab_v, [iv + (c * TPAD)])
                    obuf[c, pl.ds(p, L)] = vals

        def out_slab(cb, g):
            return out_hbm.at[pl.ds(cb * CB, CB), pl.ds(base + g * G, G)]

        def cb_body(cb, carry):
            pltpu.sync_copy(ftT_hbm.at[pl.ds(cb * (CB * TPAD), CB * TPAD)], tab_v)

            def gp_body(gp, carry2):
                g0 = 2 * gp
                g1 = 2 * gp + 1

                @pl.when(gp > 0)
                def _():
                    pltpu.make_async_copy(obA, out_slab(cb, g0), semA).wait()

                fill(obA, g0)
                pltpu.async_copy(obA, out_slab(cb, g0), semA)

                @pl.when(gp > 0)
                def _():
                    pltpu.make_async_copy(obB, out_slab(cb, g1), semB).wait()

                fill(obB, g1)
                pltpu.async_copy(obB, out_slab(cb, g1), semB)
                return carry2

            lax.fori_loop(0, NG // 2, gp_body, 0)
            pltpu.make_async_copy(obA, out_slab(cb, 0), semA).wait()
            pltpu.make_async_copy(obB, out_slab(cb, 0), semB).wait()
            return carry

        lax.fori_loop(0, NCB, cb_body, 0)

    return k(ftT, seg)


def kernel(seg_map, feature_map):
    ftT = _transpose_pad(feature_map).reshape(-1)
    out, mask = _sc_gather(ftT, seg_map.reshape(-1))
    return out.reshape(D, H, W), mask.reshape(1, H, W)


# speedup vs baseline: 3.1924x; 1.0001x over previous
import functools

import jax
import jax.numpy as jnp
from jax import lax
from jax.experimental import pallas as pl
from jax.experimental.pallas import tpu as pltpu
from jax.experimental.pallas import tpu_sc as plsc

H = 512
W = 512
P = H * W
R = 2048
D = 256
TPAD = 2056
CB = 16
NCB = D // CB
G = 1024
L = 16


def _transpose_pad(fm):
    def body(f_ref, o_ref):
        t = f_ref[...].T
        o_ref[...] = jnp.concatenate(
            [t, jnp.zeros((D, TPAD - R), jnp.float32)], axis=1
        )

    return pl.pallas_call(
        body,
        out_shape=jax.ShapeDtypeStruct((D, TPAD), jnp.float32),
    )(fm)


def _sc_gather(ftT, seg):
    info = plsc.get_sparse_core_info()
    NC, NS = info.num_cores, info.num_subcores
    NW = NC * NS
    PW = P // NW
    NG = PW // G

    mesh = plsc.VectorSubcoreMesh(core_axis_name="c", subcore_axis_name="s")

    @functools.partial(
        pl.kernel,
        mesh=mesh,
        compiler_params=pltpu.CompilerParams(
            needs_layout_passes=False, use_tc_tiling_on_sc=True
        ),
        out_type=[
            jax.ShapeDtypeStruct((D, P), jnp.float32),
            jax.ShapeDtypeStruct((P,), jnp.float32),
        ],
        scratch_types=[
            pltpu.VMEM((PW,), jnp.int32),
            pltpu.VMEM((PW,), jnp.float32),
            pltpu.VMEM((CB * TPAD,), jnp.float32),
            pltpu.VMEM((CB, G), jnp.float32),
            pltpu.VMEM((CB, G), jnp.float32),
            pltpu.SemaphoreType.DMA,
            pltpu.SemaphoreType.DMA,
        ],
    )
    def k(ftT_hbm, seg_hbm, out_hbm, mask_hbm, idx_v, val_v, tab_v,
          obA, obB, semA, semB):
        wid = lax.axis_index("s") * NC + lax.axis_index("c")
        base = wid * PW

        pltpu.sync_copy(seg_hbm.at[pl.ds(base, PW)], idx_v)

        @plsc.parallel_loop(0, PW, step=L, unroll=4)
        def _(p):
            s = idx_v[pl.ds(p, L)]
            valid = (s >= 0) & (s < R)
            idx_v[pl.ds(p, L)] = jnp.where(valid, s, R)
            val_v[pl.ds(p, L)] = jnp.where(valid, 1.0, 0.0).astype(jnp.float32)
        pltpu.sync_copy(val_v, mask_hbm.at[pl.ds(base, PW)])

        def fill(obuf, g):
            @plsc.parallel_loop(0, G, step=L, unroll=2)
            def _(p):
                iv = idx_v[pl.ds(g * G + p, L)]
                for c in range(CB):
                    vals = plsc.load_gather(tab_v, [iv + (c * TPAD)])
                    obuf[c, pl.ds(p, L)] = vals

        def out_slab(cb, g):
            return out_hbm.at[pl.ds(cb * CB, CB), pl.ds(base + g * G, G)]

        def cb_body(cb, carry):
            pltpu.sync_copy(ftT_hbm.at[pl.ds(cb * (CB * TPAD), CB * TPAD)], tab_v)

            def gp_body(gp, carry2):
                g0 = 2 * gp
                g1 = 2 * gp + 1

                @pl.when(gp > 0)
                def _():
                    pltpu.make_async_copy(obA, out_slab(cb, g0), semA).wait()

                fill(obA, g0)
                pltpu.async_copy(obA, out_slab(cb, g0), semA)

                @pl.when(gp > 0)
                def _():
                    pltpu.make_async_copy(obB, out_slab(cb, g1), semB).wait()

                fill(obB, g1)
                pltpu.async_copy(obB, out_slab(cb, g1), semB)
                return carry2

            lax.fori_loop(0, NG // 2, gp_body, 0)
            pltpu.make_async_copy(obA, out_slab(cb, 0), semA).wait()
            pltpu.make_async_copy(obB, out_slab(cb, 0), semB).wait()
            return carry

        lax.fori_loop(0, NCB, cb_body, 0)

    return k(ftT, seg)


def kernel(seg_map, feature_map):
    ftT = _transpose_pad(feature_map).reshape(-1)
    out, mask = _sc_gather(ftT, seg_map.reshape(-1))
    return out.reshape(D, H, W), mask.reshape(1, H, W)


# speedup vs baseline: 5.7835x; 1.8116x over previous
import functools

import jax
import jax.numpy as jnp
from jax import lax
from jax.experimental import pallas as pl
from jax.experimental.pallas import tpu as pltpu
from jax.experimental.pallas import tpu_sc as plsc

H = 512
W = 512
P = H * W
R = 2048
D = 256
TPAD = 2056
CB = 16
NCB = D // CB
G = W
L = 16


def _transpose_pad(fm):
    def body(f_ref, o_ref):
        t = f_ref[...].T
        o_ref[...] = jnp.concatenate(
            [t, jnp.zeros((D, TPAD - R), jnp.float32)], axis=1
        )

    return pl.pallas_call(
        body,
        out_shape=jax.ShapeDtypeStruct((D, TPAD), jnp.float32),
    )(fm)


def _sc_gather(ftT, seg):
    info = plsc.get_sparse_core_info()
    NC, NS = info.num_cores, info.num_subcores
    NW = NC * NS
    PW = P // NW
    NG = PW // G

    mesh = plsc.VectorSubcoreMesh(core_axis_name="c", subcore_axis_name="s")

    @functools.partial(
        pl.kernel,
        mesh=mesh,
        compiler_params=pltpu.CompilerParams(
            needs_layout_passes=False, use_tc_tiling_on_sc=True
        ),
        out_type=[
            jax.ShapeDtypeStruct((D, H, W), jnp.float32),
            jax.ShapeDtypeStruct((1, H, W), jnp.float32),
        ],
        scratch_types=[
            pltpu.VMEM((PW,), jnp.int32),
            pltpu.VMEM((PW,), jnp.float32),
            pltpu.VMEM((CB * TPAD,), jnp.float32),
            pltpu.VMEM((CB, G), jnp.float32),
            pltpu.VMEM((CB, G), jnp.float32),
            pltpu.SemaphoreType.DMA,
            pltpu.SemaphoreType.DMA,
        ],
    )
    def k(ftT_hbm, seg_hbm, out_hbm, mask_hbm, idx_v, val_v, tab_v,
          obA, obB, semA, semB):
        wid = lax.axis_index("s") * NC + lax.axis_index("c")
        base = wid * PW
        y0 = wid * (PW // W)

        pltpu.sync_copy(seg_hbm.at[pl.ds(base, PW)], idx_v)

        @plsc.parallel_loop(0, PW, step=L, unroll=4)
        def _(p):
            s = idx_v[pl.ds(p, L)]
            valid = (s >= 0) & (s < R)
            idx_v[pl.ds(p, L)] = jnp.where(valid, s, R)
            val_v[pl.ds(p, L)] = jnp.where(valid, 1.0, 0.0).astype(jnp.float32)
        for j in range(PW // W):
            pltpu.sync_copy(val_v.at[pl.ds(j * W, W)], mask_hbm.at[0, y0 + j, :])

        def fill(obuf, g):
            @plsc.parallel_loop(0, G, step=L, unroll=2)
            def _(p):
                iv = idx_v[pl.ds(g * G + p, L)]
                for c in range(CB):
                    vals = plsc.load_gather(tab_v, [iv + (c * TPAD)])
                    obuf[c, pl.ds(p, L)] = vals

        def out_slab(cb, g):
            return out_hbm.at[pl.ds(cb * CB, CB), y0 + g, :]

        def cb_body(cb, carry):
            pltpu.sync_copy(ftT_hbm.at[pl.ds(cb * (CB * TPAD), CB * TPAD)], tab_v)

            def gp_body(gp, carry2):
                g0 = 2 * gp
                g1 = 2 * gp + 1

                @pl.when(gp > 0)
                def _():
                    pltpu.make_async_copy(obA, out_slab(cb, g0), semA).wait()

                fill(obA, g0)
                pltpu.async_copy(obA, out_slab(cb, g0), semA)

                @pl.when(gp > 0)
                def _():
                    pltpu.make_async_copy(obB, out_slab(cb, g1), semB).wait()

                fill(obB, g1)
                pltpu.async_copy(obB, out_slab(cb, g1), semB)
                return carry2

            lax.fori_loop(0, NG // 2, gp_body, 0)
            pltpu.make_async_copy(obA, out_slab(cb, 0), semA).wait()
            pltpu.make_async_copy(obB, out_slab(cb, 0), semB).wait()
            return carry

        lax.fori_loop(0, NCB, cb_body, 0)

    return k(ftT, seg)


def kernel(seg_map, feature_map):
    ftT = _transpose_pad(feature_map).reshape(-1)
    out, mask = _sc_gather(ftT, seg_map.reshape(-1))
    return out, mask
